# baseline (speedup 1.0000x reference)
import jax
import jax.numpy as jnp
from jax import lax
from jax.experimental import pallas as pl
from jax.experimental.pallas import tpu as pltpu
from jax.experimental.pallas import tpu_sc as plsc

_NS = 8
_B, _C, _H, _W = 8, 96, 224, 224
_NCHUNK = _H // 16
_NCORES = 2
_NSUB = 16
_MAPS_PER_W = (_B * _C) // (_NCORES * _NSUB)
_CAP = _MAPS_PER_W * _H
_GK = 32


def _stats_body(x_ref, rowmax_ref, gmax_ref, f_ref):
    xb = x_ref[...]
    rowmax = jnp.max(xb, axis=2)
    rowmax_ref[...] = rowmax
    gmax_ref[...] = jnp.max(rowmax, axis=1).reshape(1, 1, -1)
    f_ref[...] = (jnp.sum(jnp.sum(xb, axis=2), axis=1) / _W).reshape(1, 1, -1)


def _splat_f(v):
    return jnp.full((16,), v, jnp.float32)


def _lane_sum(v, iota):
    for s in (1, 2, 4, 8):
        v = v + jnp.take(v, iota ^ s, mode="wrap")
    return v


def _lane_prefix(v, iota, zero):
    for s in (1, 2, 4, 8):
        w = jnp.take(v, jnp.maximum(iota - s, 0), mode="wrap")
        v = v + jnp.where(iota >= s, w, zero)
    return v


def _lane_min(v, iota):
    for s in (1, 2, 4, 8):
        v = jnp.minimum(v, jnp.take(v, iota ^ s, mode="wrap"))
    return v


def _sc_finish_body(
    x3_hbm, rowmax_hbm, gmax_hbm, f_hbm, out_hbm,
    rm_buf, gm_buf, ridx_f, tidx_f, gmlist, rows_v,
    hist, hist4, fv, outb, shared, sem,
):
    cid = lax.axis_index("c")
    sid = lax.axis_index("s")
    g0 = cid * (_NSUB * _MAPS_PER_W) + sid * _MAPS_PER_W

    pltpu.sync_copy(rowmax_hbm.at[pl.ds(g0, _MAPS_PER_W)], rm_buf)
    pltpu.sync_copy(
        gmax_hbm.at[pl.ds(g0, _MAPS_PER_W)], gm_buf.at[pl.ds(0, _MAPS_PER_W)]
    )

    zero16f = jnp.zeros((16,), jnp.float32)
    zero16i = jnp.zeros((16,), jnp.int32)
    one16f = jnp.ones((16,), jnp.float32)
    for i in range(_NCHUNK + 1):
        hist[pl.ds(i * 16, 16)] = zero16f

    def zero_body(i, _):
        ridx_f[pl.ds(i * 16, 16)] = zero16f
        tidx_f[pl.ds(i * 16, 16)] = zero16f
        return _

    lax.fori_loop(0, (_CAP + 32) // 16, zero_body, 0)

    iota16 = lax.iota(jnp.int32, 16)
    iota_f = iota16.astype(jnp.float32)
    big16f = jnp.full((16,), 1.0e9, jnp.float32)

    def map_body(m, off):
        gm_v = _splat_f(gm_buf[pl.ds(m, 16)][0])
        rbase_f = ((g0 + m) * _H).astype(jnp.float32)
        tbase_f = ((g0 + m) * (_H // 8)).astype(jnp.float32)
        for i in range(_NCHUNK):
            rm_c = rm_buf[m, pl.ds(i * 16, 16)]
            msk = rm_c == gm_v
            mi = jnp.where(msk, one16f, zero16f)
            nh = _lane_sum(mi, iota16)[0].astype(jnp.int32)

            def hit_body(_hh, carry):
                mi2, off2 = carry
                lane = jnp.where(mi2 > 0.0, iota_f, big16f)
                h0 = _lane_min(lane, iota16)[0]
                ridx_f[pl.ds(off2, 16)] = _splat_f(
                    rbase_f + float(i * 16) + h0
                )
                tidx_f[pl.ds(off2, 16)] = _splat_f(
                    tbase_f
                    + float(2 * i)
                    + jnp.where(h0 >= 8.0, 1.0, 0.0)
                )
                gmlist[pl.ds(off2, 16)] = gm_v
                mi2 = jnp.where(iota_f == _splat_f(h0), zero16f, mi2)
                return (mi2, off2 + 1)

            _, off = lax.fori_loop(0, nh, hit_body, (mi, off))
        return off

    n = lax.fori_loop(0, _MAPS_PER_W, map_body, jnp.int32(0))

    def row_loop(q, _):
        r_i = ridx_f[pl.ds(q, 16)][0].astype(jnp.int32)
        t_i = tidx_f[pl.ds(q, 16)][0].astype(jnp.int32)
        gm_v = _splat_f(gmlist[pl.ds(q, 16)][0])
        pltpu.sync_copy(x3_hbm.at[t_i], rows_v)
        h_r = lax.rem(r_i, _H)
        hsub = lax.rem(h_r, 8)
        cnt = zero16f
        for i in range(_NCHUNK):
            xc = rows_v[hsub, pl.ds(i * 16, 16)]
            cnt = cnt + jnp.where(xc == gm_v, one16f, zero16f)
        tot = _lane_sum(cnt, iota16)
        hv = hist[pl.ds(h_r, 16)]
        hist[pl.ds(h_r, 16)] = hv + jnp.where(iota16 == 0, tot, 0.0)
        return _

    lax.fori_loop(0, n, row_loop, 0)

    pltpu.sync_copy(hist.at[pl.ds(0, _H)], shared.at[pl.ds(sid * _H, _H)])
    plsc.subcore_barrier()

    @pl.when(sid < _B // _NCORES)
    def _finalize():
        b = cid * (_B // _NCORES) + sid
        pltpu.sync_copy(shared.at[pl.ds(sid * 4 * _H, 4 * _H)], hist4)
        pltpu.sync_copy(f_hbm.at[pl.ds(b * _C, _C)], fv)

        jcnt = [zero16f] * _NS
        thr = [_splat_f(float((k * _C) // _NS)) for k in range(_NS)]

        def chunk_body(i, carry):
            carry_s, *jc = carry
            v = (
                hist4[pl.ds(i * 16, 16)]
                + hist4[pl.ds(_H + i * 16, 16)]
                + hist4[pl.ds(2 * _H + i * 16, 16)]
                + hist4[pl.ds(3 * _H + i * 16, 16)]
            )
            cs = _lane_prefix(v, iota16, zero16f)
            hexc = cs - v + _splat_f(carry_s)
            jc = list(jc)
            for k in range(1, _NS):
                jc[k] = jc[k] + jnp.where(hexc <= thr[k], one16f, zero16f)
            carry_s = carry_s + cs[15]
            return (carry_s, *jc)

        res = lax.fori_loop(
            0, _NCHUNK, chunk_body, (jnp.float32(0.0), *jcnt)
        )
        jcnt = [
            _lane_sum(v, iota16) if k else v for k, v in enumerate(res[1:])
        ]

        one_f = jnp.ones((16,), jnp.float32)
        prev = jnp.zeros((16,), jnp.float32)
        ok = jnp.ones((16,), jnp.bool_)
        hk = [jnp.zeros((16,), jnp.float32)]
        for k in range(1, _NS):
            jk = jcnt[k] - one_f
            good = (jk >= one_f) & (jk <= float(_H - 2)) & (jk > prev)
            ok = ok & good
            hk.append(jnp.where(ok, jk, 0.0))
            prev = jk
        hk.append(_splat_f(float(_H)))

        for s in range(_NS):
            d_v = hk[s + 1] - hk[s]
            for ci in range(_C // 16):
                outb[pl.ds(s * _C + ci * 16, 16)] = fv[pl.ds(ci * 16, 16)] / d_v

        pltpu.sync_copy(outb, out_hbm.at[b])


def _sc_finish(x2, rowmax, gmax, f):
    mesh = plsc.VectorSubcoreMesh(
        core_axis_name="c", subcore_axis_name="s",
        num_cores=_NCORES, num_subcores=_NSUB,
    )
    kfun = pl.kernel(
        _sc_finish_body,
        out_type=jax.ShapeDtypeStruct((_B, _NS * _C), jnp.float32),
        mesh=mesh,
        scratch_types=[
            pltpu.VMEM((_MAPS_PER_W, _H), jnp.float32),
            pltpu.VMEM((_MAPS_PER_W + 16,), jnp.float32),
            pltpu.VMEM((_CAP + 32,), jnp.float32),
            pltpu.VMEM((_CAP + 32,), jnp.float32),
            pltpu.VMEM((_CAP + 16,), jnp.float32),
            pltpu.VMEM((8, _W), jnp.float32),
            pltpu.VMEM((_H + 16,), jnp.float32),
            pltpu.VMEM((4 * _H,), jnp.float32),
            pltpu.VMEM((_C,), jnp.float32),
            pltpu.VMEM((_NS * _C,), jnp.float32),
            pltpu.VMEM_SHARED((_NSUB * _H,), jnp.float32),
            pltpu.SemaphoreType.DMA,
        ],
    )
    return kfun(x2, rowmax, gmax, f)


def kernel(x):
    B, C, H, W = x.shape
    x3 = x.reshape(B * C, H, W)
    CB = 8
    rowmax, gmax3, f3 = pl.pallas_call(
        _stats_body,
        grid=(B * C // CB,),
        in_specs=[pl.BlockSpec((CB, H, W), lambda g: (g, 0, 0))],
        out_specs=[
            pl.BlockSpec((CB, H), lambda g: (g, 0)),
            pl.BlockSpec((1, 1, CB), lambda g: (g, 0, 0)),
            pl.BlockSpec((1, 1, CB), lambda g: (g, 0, 0)),
        ],
        out_shape=[
            jax.ShapeDtypeStruct((B * C, H), jnp.float32),
            jax.ShapeDtypeStruct((B * C // CB, 1, CB), jnp.float32),
            jax.ShapeDtypeStruct((B * C // CB, 1, CB), jnp.float32),
        ],
    )(x3)
    return _sc_finish(
        x.reshape(B * C * H // 8, 8, W),
        rowmax,
        gmax3.reshape(B * C),
        f3.reshape(B * C),
    )


# speedup vs baseline: 1.2135x; 1.2135x over previous
import jax
import jax.numpy as jnp
from jax import lax
from jax.experimental import pallas as pl
from jax.experimental.pallas import tpu as pltpu
from jax.experimental.pallas import tpu_sc as plsc

_NS = 8
_B, _C, _H, _W = 8, 96, 224, 224
_NCHUNK = _H // 16
_NCORES = 2
_NSUB = 16
_MAPS_PER_W = (_B * _C) // (_NCORES * _NSUB)


def _stats_body(x_ref, rowmax_ref, gmax_ref, f_ref, hrow_ref, nties_ref):
    xb = x_ref[...]
    rowmax = jnp.max(xb, axis=2)
    rowmax_ref[...] = rowmax
    gmax = jnp.max(rowmax, axis=1)
    gmax_ref[...] = gmax.reshape(1, 1, -1)
    f_ref[...] = (jnp.sum(jnp.sum(xb, axis=2), axis=1) / _W).reshape(1, 1, -1)
    hrow_ref[...] = (
        jnp.argmax(rowmax, axis=1).astype(jnp.float32).reshape(1, 1, -1)
    )
    nties_ref[...] = jnp.sum(
        jnp.where(rowmax == gmax[:, None], 1.0, 0.0), axis=1
    ).reshape(1, 1, -1)


def _splat_f(v):
    return jnp.full((16,), v, jnp.float32)


def _lane_sum(v, iota):
    for s in (1, 2, 4, 8):
        v = v + jnp.take(v, iota ^ s, mode="wrap")
    return v


def _lane_prefix(v, iota, zero):
    for s in (1, 2, 4, 8):
        w = jnp.take(v, jnp.maximum(iota - s, 0), mode="wrap")
        v = v + jnp.where(iota >= s, w, zero)
    return v


def _lane_min(v, iota):
    for s in (1, 2, 4, 8):
        v = jnp.minimum(v, jnp.take(v, iota ^ s, mode="wrap"))
    return v


def _lane_max(v, iota):
    for s in (1, 2, 4, 8):
        v = jnp.maximum(v, jnp.take(v, iota ^ s, mode="wrap"))
    return v


def _sc_finish_body(
    x3_hbm, rowmax_hbm, gmax_hbm, f_hbm, hrow_hbm, nties_hbm, out_hbm,
    rm_buf, gm_buf, hrow_buf, nties_buf, rows_v,
    hist, hist4, fv, outb, shared,
):
    cid = lax.axis_index("c")
    sid = lax.axis_index("s")
    g0 = cid * (_NSUB * _MAPS_PER_W) + sid * _MAPS_PER_W

    pltpu.sync_copy(
        gmax_hbm.at[pl.ds(g0, _MAPS_PER_W)], gm_buf.at[pl.ds(0, _MAPS_PER_W)]
    )
    pltpu.sync_copy(
        hrow_hbm.at[pl.ds(g0, _MAPS_PER_W)], hrow_buf.at[pl.ds(0, _MAPS_PER_W)]
    )
    pltpu.sync_copy(
        nties_hbm.at[pl.ds(g0, _MAPS_PER_W)],
        nties_buf.at[pl.ds(0, _MAPS_PER_W)],
    )

    zero16f = jnp.zeros((16,), jnp.float32)
    one16f = jnp.ones((16,), jnp.float32)
    for i in range(_NCHUNK + 1):
        hist[pl.ds(i * 16, 16)] = zero16f

    iota16 = lax.iota(jnp.int32, 16)
    iota_f = iota16.astype(jnp.float32)
    big16f = jnp.full((16,), 1.0e9, jnp.float32)

    nta = jnp.maximum(nties_buf[pl.ds(0, 16)], nties_buf[pl.ds(8, 16)])
    any_slow = _lane_max(nta, iota16)[0] > 1.0

    @pl.when(any_slow)
    def _load_rowmax():
        pltpu.sync_copy(rowmax_hbm.at[pl.ds(g0, _MAPS_PER_W)], rm_buf)

    def _count_row(h_i, gm_v, tile_i):
        pltpu.sync_copy(x3_hbm.at[tile_i], rows_v)
        hsub = lax.rem(h_i, 8)
        cnt = zero16f
        for i in range(_NCHUNK):
            xc = rows_v[hsub, pl.ds(i * 16, 16)]
            cnt = cnt + jnp.where(xc == gm_v, one16f, zero16f)
        tot = _lane_sum(cnt, iota16)
        hv = hist[pl.ds(h_i, 16)]
        hist[pl.ds(h_i, 16)] = hv + jnp.where(iota16 == 0, tot, 0.0)

    def map_body(m, _):
        gm_v = _splat_f(gm_buf[pl.ds(m, 16)][0])
        nt = nties_buf[pl.ds(m, 16)][0]
        tbase = (g0 + m) * (_H // 8)

        @pl.when(nt == 1.0)
        def _fast():
            h_i = hrow_buf[pl.ds(m, 16)][0].astype(jnp.int32)
            _count_row(h_i, gm_v, tbase + lax.div(h_i, 8))

        @pl.when(nt > 1.0)
        def _slow():
            for i in range(_NCHUNK):
                rm_c = rm_buf[m, pl.ds(i * 16, 16)]
                mi = jnp.where(rm_c == gm_v, one16f, zero16f)
                nh = _lane_sum(mi, iota16)[0].astype(jnp.int32)

                def hit_body(_hh, mi2):
                    lane = jnp.where(mi2 > 0.0, iota_f, big16f)
                    h0 = _lane_min(lane, iota16)[0]
                    h_i = h0.astype(jnp.int32) + i * 16
                    _count_row(h_i, gm_v, tbase + lax.div(h_i, 8))
                    return jnp.where(iota_f == _splat_f(h0), zero16f, mi2)

                lax.fori_loop(0, nh, hit_body, mi)

        return _

    lax.fori_loop(0, _MAPS_PER_W, map_body, 0)

    pltpu.sync_copy(hist.at[pl.ds(0, _H)], shared.at[pl.ds(sid * _H, _H)])
    plsc.subcore_barrier()

    @pl.when(sid < _B // _NCORES)
    def _finalize():
        b = cid * (_B // _NCORES) + sid
        pltpu.sync_copy(shared.at[pl.ds(sid * 4 * _H, 4 * _H)], hist4)
        pltpu.sync_copy(f_hbm.at[pl.ds(b * _C, _C)], fv)

        jcnt = [zero16f] * _NS
        thr = [_splat_f(float((k * _C) // _NS)) for k in range(_NS)]

        def chunk_body(i, carry):
            carry_s, *jc = carry
            v = (
                hist4[pl.ds(i * 16, 16)]
                + hist4[pl.ds(_H + i * 16, 16)]
                + hist4[pl.ds(2 * _H + i * 16, 16)]
                + hist4[pl.ds(3 * _H + i * 16, 16)]
            )
            cs = _lane_prefix(v, iota16, zero16f)
            hexc = cs - v + _splat_f(carry_s)
            jc = list(jc)
            for k in range(1, _NS):
                jc[k] = jc[k] + jnp.where(hexc <= thr[k], one16f, zero16f)
            carry_s = carry_s + cs[15]
            return (carry_s, *jc)

        res = lax.fori_loop(
            0, _NCHUNK, chunk_body, (jnp.float32(0.0), *jcnt)
        )
        jcnt = [
            _lane_sum(v, iota16) if k else v for k, v in enumerate(res[1:])
        ]

        one_f = jnp.ones((16,), jnp.float32)
        prev = jnp.zeros((16,), jnp.float32)
        ok = jnp.ones((16,), jnp.bool_)
        hk = [jnp.zeros((16,), jnp.float32)]
        for k in range(1, _NS):
            jk = jcnt[k] - one_f
            good = (jk >= one_f) & (jk <= float(_H - 2)) & (jk > prev)
            ok = ok & good
            hk.append(jnp.where(ok, jk, 0.0))
            prev = jk
        hk.append(_splat_f(float(_H)))

        for s in range(_NS):
            d_v = hk[s + 1] - hk[s]
            for ci in range(_C // 16):
                outb[pl.ds(s * _C + ci * 16, 16)] = fv[pl.ds(ci * 16, 16)] / d_v

        pltpu.sync_copy(outb, out_hbm.at[b])


def _sc_finish(x3, rowmax, gmax, f, hrow, nties):
    mesh = plsc.VectorSubcoreMesh(
        core_axis_name="c", subcore_axis_name="s",
        num_cores=_NCORES, num_subcores=_NSUB,
    )
    kfun = pl.kernel(
        _sc_finish_body,
        out_type=jax.ShapeDtypeStruct((_B, _NS * _C), jnp.float32),
        mesh=mesh,
        scratch_types=[
            pltpu.VMEM((_MAPS_PER_W, _H), jnp.float32),
            pltpu.VMEM((_MAPS_PER_W + 16,), jnp.float32),
            pltpu.VMEM((_MAPS_PER_W + 16,), jnp.float32),
            pltpu.VMEM((_MAPS_PER_W + 16,), jnp.float32),
            pltpu.VMEM((8, _W), jnp.float32),
            pltpu.VMEM((_H + 16,), jnp.float32),
            pltpu.VMEM((4 * _H,), jnp.float32),
            pltpu.VMEM((_C,), jnp.float32),
            pltpu.VMEM((_NS * _C,), jnp.float32),
            pltpu.VMEM_SHARED((_NSUB * _H,), jnp.float32),
        ],
    )
    return kfun(x3, rowmax, gmax, f, hrow, nties)


def kernel(x):
    B, C, H, W = x.shape
    x3 = x.reshape(B * C, H, W)
    CB = 8
    rowmax, gmax3, f3, hrow3, nties3 = pl.pallas_call(
        _stats_body,
        grid=(B * C // CB,),
        in_specs=[pl.BlockSpec((CB, H, W), lambda g: (g, 0, 0))],
        out_specs=[
            pl.BlockSpec((CB, H), lambda g: (g, 0)),
            pl.BlockSpec((1, 1, CB), lambda g: (g, 0, 0)),
            pl.BlockSpec((1, 1, CB), lambda g: (g, 0, 0)),
            pl.BlockSpec((1, 1, CB), lambda g: (g, 0, 0)),
            pl.BlockSpec((1, 1, CB), lambda g: (g, 0, 0)),
        ],
        out_shape=[
            jax.ShapeDtypeStruct((B * C, H), jnp.float32),
            jax.ShapeDtypeStruct((B * C // CB, 1, CB), jnp.float32),
            jax.ShapeDtypeStruct((B * C // CB, 1, CB), jnp.float32),
            jax.ShapeDtypeStruct((B * C // CB, 1, CB), jnp.float32),
            jax.ShapeDtypeStruct((B * C // CB, 1, CB), jnp.float32),
        ],
    )(x3)
    return _sc_finish(
        x.reshape(B * C * H // 8, 8, W),
        rowmax,
        gmax3.reshape(B * C),
        f3.reshape(B * C),
        hrow3.reshape(B * C),
        nties3.reshape(B * C),
    )


# speedup vs baseline: 1.3128x; 1.0819x over previous
import jax
import jax.numpy as jnp
from jax import lax
from jax.experimental import pallas as pl
from jax.experimental.pallas import tpu as pltpu
from jax.experimental.pallas import tpu_sc as plsc

_NS = 8
_B, _C, _H, _W = 8, 96, 224, 224
_NCHUNK = _H // 16
_NCORES = 2
_NSUB = 16
_MAPS_PER_W = (_B * _C) // (_NCORES * _NSUB)


def _stats_body(x_ref, rowmax_ref, gmax_ref, f_ref, hrow_ref, nties_ref):
    cb = x_ref.shape[0]
    facc = jnp.zeros((cb, 8), jnp.float32)
    for t in range(_H // 8):
        xt = x_ref[:, pl.ds(t * 8, 8), :]
        rowmax_ref[:, pl.ds(t * 8, 8)] = jnp.max(xt, axis=2)
        facc = facc + jnp.sum(xt, axis=2)
    f_ref[...] = (jnp.sum(facc, axis=1) / _W).reshape(1, 1, -1)
    rowmax = rowmax_ref[...]
    gmax = jnp.max(rowmax, axis=1)
    gmax_ref[...] = gmax.reshape(1, 1, -1)
    hrow_ref[...] = (
        jnp.argmax(rowmax, axis=1).astype(jnp.float32).reshape(1, 1, -1)
    )
    nties_ref[...] = jnp.sum(
        jnp.where(rowmax == gmax[:, None], 1.0, 0.0), axis=1
    ).reshape(1, 1, -1)


def _splat_f(v):
    return jnp.full((16,), v, jnp.float32)


def _lane_sum(v, iota):
    for s in (1, 2, 4, 8):
        v = v + jnp.take(v, iota ^ s, mode="wrap")
    return v


def _lane_prefix(v, iota, zero):
    for s in (1, 2, 4, 8):
        w = jnp.take(v, jnp.maximum(iota - s, 0), mode="wrap")
        v = v + jnp.where(iota >= s, w, zero)
    return v


def _lane_min(v, iota):
    for s in (1, 2, 4, 8):
        v = jnp.minimum(v, jnp.take(v, iota ^ s, mode="wrap"))
    return v


def _lane_max(v, iota):
    for s in (1, 2, 4, 8):
        v = jnp.maximum(v, jnp.take(v, iota ^ s, mode="wrap"))
    return v


def _sc_finish_body(
    x3_hbm, rowmax_hbm, gmax_hbm, f_hbm, hrow_hbm, nties_hbm, out_hbm,
    rm_buf, gm_buf, hrow_buf, nties_buf, rows_v,
    hist, hist4, fv, outb, shared,
):
    cid = lax.axis_index("c")
    sid = lax.axis_index("s")
    g0 = cid * (_NSUB * _MAPS_PER_W) + sid * _MAPS_PER_W

    pltpu.sync_copy(
        gmax_hbm.at[pl.ds(g0, _MAPS_PER_W)], gm_buf.at[pl.ds(0, _MAPS_PER_W)]
    )
    pltpu.sync_copy(
        hrow_hbm.at[pl.ds(g0, _MAPS_PER_W)], hrow_buf.at[pl.ds(0, _MAPS_PER_W)]
    )
    pltpu.sync_copy(
        nties_hbm.at[pl.ds(g0, _MAPS_PER_W)],
        nties_buf.at[pl.ds(0, _MAPS_PER_W)],
    )

    zero16f = jnp.zeros((16,), jnp.float32)
    one16f = jnp.ones((16,), jnp.float32)
    for i in range(_NCHUNK + 1):
        hist[pl.ds(i * 16, 16)] = zero16f

    iota16 = lax.iota(jnp.int32, 16)
    iota_f = iota16.astype(jnp.float32)
    big16f = jnp.full((16,), 1.0e9, jnp.float32)

    nta = jnp.maximum(nties_buf[pl.ds(0, 16)], nties_buf[pl.ds(8, 16)])
    any_slow = _lane_max(nta, iota16)[0] > 1.0

    @pl.when(any_slow)
    def _load_rowmax():
        pltpu.sync_copy(rowmax_hbm.at[pl.ds(g0, _MAPS_PER_W)], rm_buf)

    def _count_row(h_i, gm_v, tile_i):
        pltpu.sync_copy(x3_hbm.at[tile_i], rows_v)
        hsub = lax.rem(h_i, 8)
        cnt = zero16f
        for i in range(_NCHUNK):
            xc = rows_v[hsub, pl.ds(i * 16, 16)]
            cnt = cnt + jnp.where(xc == gm_v, one16f, zero16f)
        tot = _lane_sum(cnt, iota16)
        hv = hist[pl.ds(h_i, 16)]
        hist[pl.ds(h_i, 16)] = hv + jnp.where(iota16 == 0, tot, 0.0)

    def map_body(m, _):
        gm_v = _splat_f(gm_buf[pl.ds(m, 16)][0])
        nt = nties_buf[pl.ds(m, 16)][0]
        tbase = (g0 + m) * (_H // 8)

        @pl.when(nt == 1.0)
        def _fast():
            h_i = hrow_buf[pl.ds(m, 16)][0].astype(jnp.int32)
            _count_row(h_i, gm_v, tbase + lax.div(h_i, 8))

        @pl.when(nt > 1.0)
        def _slow():
            for i in range(_NCHUNK):
                rm_c = rm_buf[m, pl.ds(i * 16, 16)]
                mi = jnp.where(rm_c == gm_v, one16f, zero16f)
                nh = _lane_sum(mi, iota16)[0].astype(jnp.int32)

                def hit_body(_hh, mi2):
                    lane = jnp.where(mi2 > 0.0, iota_f, big16f)
                    h0 = _lane_min(lane, iota16)[0]
                    h_i = h0.astype(jnp.int32) + i * 16
                    _count_row(h_i, gm_v, tbase + lax.div(h_i, 8))
                    return jnp.where(iota_f == _splat_f(h0), zero16f, mi2)

                lax.fori_loop(0, nh, hit_body, mi)

        return _

    lax.fori_loop(0, _MAPS_PER_W, map_body, 0)

    pltpu.sync_copy(hist.at[pl.ds(0, _H)], shared.at[pl.ds(sid * _H, _H)])
    plsc.subcore_barrier()

    @pl.when(sid < _B // _NCORES)
    def _finalize():
        b = cid * (_B // _NCORES) + sid
        pltpu.sync_copy(shared.at[pl.ds(sid * 4 * _H, 4 * _H)], hist4)
        pltpu.sync_copy(f_hbm.at[pl.ds(b * _C, _C)], fv)

        jcnt = [zero16f] * _NS
        thr = [_splat_f(float((k * _C) // _NS)) for k in range(_NS)]

        def chunk_body(i, carry):
            carry_s, *jc = carry
            v = (
                hist4[pl.ds(i * 16, 16)]
                + hist4[pl.ds(_H + i * 16, 16)]
                + hist4[pl.ds(2 * _H + i * 16, 16)]
                + hist4[pl.ds(3 * _H + i * 16, 16)]
            )
            cs = _lane_prefix(v, iota16, zero16f)
            hexc = cs - v + _splat_f(carry_s)
            jc = list(jc)
            for k in range(1, _NS):
                jc[k] = jc[k] + jnp.where(hexc <= thr[k], one16f, zero16f)
            carry_s = carry_s + cs[15]
            return (carry_s, *jc)

        res = lax.fori_loop(
            0, _NCHUNK, chunk_body, (jnp.float32(0.0), *jcnt)
        )
        jcnt = [
            _lane_sum(v, iota16) if k else v for k, v in enumerate(res[1:])
        ]

        one_f = jnp.ones((16,), jnp.float32)
        prev = jnp.zeros((16,), jnp.float32)
        ok = jnp.ones((16,), jnp.bool_)
        hk = [jnp.zeros((16,), jnp.float32)]
        for k in range(1, _NS):
            jk = jcnt[k] - one_f
            good = (jk >= one_f) & (jk <= float(_H - 2)) & (jk > prev)
            ok = ok & good
            hk.append(jnp.where(ok, jk, 0.0))
            prev = jk
        hk.append(_splat_f(float(_H)))

        for s in range(_NS):
            d_v = hk[s + 1] - hk[s]
            for ci in range(_C // 16):
                outb[pl.ds(s * _C + ci * 16, 16)] = fv[pl.ds(ci * 16, 16)] / d_v

        pltpu.sync_copy(outb, out_hbm.at[b])


def _sc_finish(x3, rowmax, gmax, f, hrow, nties):
    mesh = plsc.VectorSubcoreMesh(
        core_axis_name="c", subcore_axis_name="s",
        num_cores=_NCORES, num_subcores=_NSUB,
    )
    kfun = pl.kernel(
        _sc_finish_body,
        out_type=jax.ShapeDtypeStruct((_B, _NS * _C), jnp.float32),
        mesh=mesh,
        scratch_types=[
            pltpu.VMEM((_MAPS_PER_W, _H), jnp.float32),
            pltpu.VMEM((_MAPS_PER_W + 16,), jnp.float32),
            pltpu.VMEM((_MAPS_PER_W + 16,), jnp.float32),
            pltpu.VMEM((_MAPS_PER_W + 16,), jnp.float32),
            pltpu.VMEM((8, _W), jnp.float32),
            pltpu.VMEM((_H + 16,), jnp.float32),
            pltpu.VMEM((4 * _H,), jnp.float32),
            pltpu.VMEM((_C,), jnp.float32),
            pltpu.VMEM((_NS * _C,), jnp.float32),
            pltpu.VMEM_SHARED((_NSUB * _H,), jnp.float32),
        ],
    )
    return kfun(x3, rowmax, gmax, f, hrow, nties)


def kernel(x):
    B, C, H, W = x.shape
    x3 = x.reshape(B * C, H, W)
    CB = 8
    rowmax, gmax3, f3, hrow3, nties3 = pl.pallas_call(
        _stats_body,
        grid=(B * C // CB,),
        in_specs=[pl.BlockSpec((CB, H, W), lambda g: (g, 0, 0))],
        out_specs=[
            pl.BlockSpec((CB, H), lambda g: (g, 0)),
            pl.BlockSpec((1, 1, CB), lambda g: (g, 0, 0)),
            pl.BlockSpec((1, 1, CB), lambda g: (g, 0, 0)),
            pl.BlockSpec((1, 1, CB), lambda g: (g, 0, 0)),
            pl.BlockSpec((1, 1, CB), lambda g: (g, 0, 0)),
        ],
        out_shape=[
            jax.ShapeDtypeStruct((B * C, H), jnp.float32),
            jax.ShapeDtypeStruct((B * C // CB, 1, CB), jnp.float32),
            jax.ShapeDtypeStruct((B * C // CB, 1, CB), jnp.float32),
            jax.ShapeDtypeStruct((B * C // CB, 1, CB), jnp.float32),
            jax.ShapeDtypeStruct((B * C // CB, 1, CB), jnp.float32),
        ],
    )(x3)
    return _sc_finish(
        x.reshape(B * C * H // 8, 8, W),
        rowmax,
        gmax3.reshape(B * C),
        f3.reshape(B * C),
        hrow3.reshape(B * C),
        nties3.reshape(B * C),
    )


# speedup vs baseline: 1.5786x; 1.2024x over previous
import jax
import jax.numpy as jnp
from jax import lax
from jax.experimental import pallas as pl
from jax.experimental.pallas import tpu as pltpu
from jax.experimental.pallas import tpu_sc as plsc

_NS = 8
_B, _C, _H, _W = 8, 96, 224, 224
_NCHUNK = _H // 16
_NCORES = 2
_NSUB = 16
_MAPS_PER_W = (_B * _C) // (_NCORES * _NSUB)


def _stats_body(x_ref, rowmax_ref, gmax_ref, f_ref, hrow_ref, nties_ref):
    cb = x_ref.shape[0]
    facc = jnp.zeros((cb, 8), jnp.float32)
    for t in range(_H // 8):
        xt = x_ref[:, pl.ds(t * 8, 8), :]
        rowmax_ref[:, pl.ds(t * 8, 8)] = jnp.max(xt, axis=2)
        facc = facc + jnp.sum(xt, axis=2)
    f_ref[...] = (jnp.sum(facc, axis=1) / _W).reshape(1, 1, -1)
    rowmax = rowmax_ref[...]
    gmax = jnp.max(rowmax, axis=1)
    gmax_ref[...] = gmax.reshape(1, 1, -1)
    hrow_ref[...] = (
        jnp.argmax(rowmax, axis=1).astype(jnp.float32).reshape(1, 1, -1)
    )
    nties_ref[...] = jnp.sum(
        jnp.where(rowmax == gmax[:, None], 1.0, 0.0), axis=1
    ).reshape(1, 1, -1)


def _splat_f(v):
    return jnp.full((16,), v, jnp.float32)


def _lane_sum(v, iota):
    for s in (1, 2, 4, 8):
        v = v + jnp.take(v, iota ^ s, mode="wrap")
    return v


def _lane_prefix(v, iota, zero):
    for s in (1, 2, 4, 8):
        w = jnp.take(v, jnp.maximum(iota - s, 0), mode="wrap")
        v = v + jnp.where(iota >= s, w, zero)
    return v


def _lane_min(v, iota):
    for s in (1, 2, 4, 8):
        v = jnp.minimum(v, jnp.take(v, iota ^ s, mode="wrap"))
    return v


def _lane_max(v, iota):
    for s in (1, 2, 4, 8):
        v = jnp.maximum(v, jnp.take(v, iota ^ s, mode="wrap"))
    return v


def _sc_finish_body(
    x3_hbm, rowmax_hbm, gmax_hbm, f_hbm, hrow_hbm, nties_hbm, out_hbm,
    rm_buf, gm_buf, hrow_buf, nties_buf, rows_v,
    hist, hist4, fv, outb, shared,
):
    cid = lax.axis_index("c")
    sid = lax.axis_index("s")
    g0 = cid * (_NSUB * _MAPS_PER_W) + sid * _MAPS_PER_W

    pltpu.sync_copy(
        gmax_hbm.at[pl.ds(g0, _MAPS_PER_W)], gm_buf.at[pl.ds(0, _MAPS_PER_W)]
    )
    pltpu.sync_copy(
        hrow_hbm.at[pl.ds(g0, _MAPS_PER_W)], hrow_buf.at[pl.ds(0, _MAPS_PER_W)]
    )
    pltpu.sync_copy(
        nties_hbm.at[pl.ds(g0, _MAPS_PER_W)],
        nties_buf.at[pl.ds(0, _MAPS_PER_W)],
    )

    zero16f = jnp.zeros((16,), jnp.float32)
    one16f = jnp.ones((16,), jnp.float32)
    for i in range(_NCHUNK + 1):
        hist[pl.ds(i * 16, 16)] = zero16f

    iota16 = lax.iota(jnp.int32, 16)
    iota_f = iota16.astype(jnp.float32)
    big16f = jnp.full((16,), 1.0e9, jnp.float32)

    nta = jnp.maximum(nties_buf[pl.ds(0, 16)], nties_buf[pl.ds(8, 16)])
    any_slow = _lane_max(nta, iota16)[0] > 1.0

    @pl.when(any_slow)
    def _load_rowmax():
        pltpu.sync_copy(rowmax_hbm.at[pl.ds(g0, _MAPS_PER_W)], rm_buf)

    def _count_row(h_i, gm_v, tile_i):
        pltpu.sync_copy(x3_hbm.at[tile_i], rows_v)
        hsub = lax.rem(h_i, 8)
        cnt = zero16f
        for i in range(_NCHUNK):
            xc = rows_v[hsub, pl.ds(i * 16, 16)]
            cnt = cnt + jnp.where(xc == gm_v, one16f, zero16f)
        tot = _lane_sum(cnt, iota16)
        hv = hist[pl.ds(h_i, 16)]
        hist[pl.ds(h_i, 16)] = hv + jnp.where(iota16 == 0, tot, 0.0)

    def map_body(m, _):
        gm_v = _splat_f(gm_buf[pl.ds(m, 16)][0])
        nt = nties_buf[pl.ds(m, 16)][0]
        tbase = (g0 + m) * (_H // 8)

        @pl.when(nt == 1.0)
        def _fast():
            h_i = hrow_buf[pl.ds(m, 16)][0].astype(jnp.int32)
            _count_row(h_i, gm_v, tbase + lax.div(h_i, 8))

        @pl.when(nt > 1.0)
        def _slow():
            for i in range(_NCHUNK):
                rm_c = rm_buf[m, pl.ds(i * 16, 16)]
                mi = jnp.where(rm_c == gm_v, one16f, zero16f)
                nh = _lane_sum(mi, iota16)[0].astype(jnp.int32)

                def hit_body(_hh, mi2):
                    lane = jnp.where(mi2 > 0.0, iota_f, big16f)
                    h0 = _lane_min(lane, iota16)[0]
                    h_i = h0.astype(jnp.int32) + i * 16
                    _count_row(h_i, gm_v, tbase + lax.div(h_i, 8))
                    return jnp.where(iota_f == _splat_f(h0), zero16f, mi2)

                lax.fori_loop(0, nh, hit_body, mi)

        return _

    lax.fori_loop(0, _MAPS_PER_W, map_body, 0)

    pltpu.sync_copy(hist.at[pl.ds(0, _H)], shared.at[pl.ds(sid * _H, _H)])
    plsc.subcore_barrier()

    @pl.when(sid < _B // _NCORES)
    def _finalize():
        b = cid * (_B // _NCORES) + sid
        pltpu.sync_copy(shared.at[pl.ds(sid * 4 * _H, 4 * _H)], hist4)
        pltpu.sync_copy(f_hbm.at[pl.ds(b * _C, _C)], fv)

        jcnt = [zero16f] * _NS
        thr = [_splat_f(float((k * _C) // _NS)) for k in range(_NS)]

        def chunk_body(i, carry):
            carry_s, *jc = carry
            v = (
                hist4[pl.ds(i * 16, 16)]
                + hist4[pl.ds(_H + i * 16, 16)]
                + hist4[pl.ds(2 * _H + i * 16, 16)]
                + hist4[pl.ds(3 * _H + i * 16, 16)]
            )
            cs = _lane_prefix(v, iota16, zero16f)
            hexc = cs - v + _splat_f(carry_s)
            jc = list(jc)
            for k in range(1, _NS):
                jc[k] = jc[k] + jnp.where(hexc <= thr[k], one16f, zero16f)
            carry_s = carry_s + cs[15]
            return (carry_s, *jc)

        res = lax.fori_loop(
            0, _NCHUNK, chunk_body, (jnp.float32(0.0), *jcnt)
        )
        jcnt = [
            _lane_sum(v, iota16) if k else v for k, v in enumerate(res[1:])
        ]

        one_f = jnp.ones((16,), jnp.float32)
        prev = jnp.zeros((16,), jnp.float32)
        ok = jnp.ones((16,), jnp.bool_)
        hk = [jnp.zeros((16,), jnp.float32)]
        for k in range(1, _NS):
            jk = jcnt[k] - one_f
            good = (jk >= one_f) & (jk <= float(_H - 2)) & (jk > prev)
            ok = ok & good
            hk.append(jnp.where(ok, jk, 0.0))
            prev = jk
        hk.append(_splat_f(float(_H)))

        for s in range(_NS):
            d_v = hk[s + 1] - hk[s]
            for ci in range(_C // 16):
                outb[pl.ds(s * _C + ci * 16, 16)] = fv[pl.ds(ci * 16, 16)] / d_v

        pltpu.sync_copy(outb, out_hbm.at[b])


def _sc_finish(x3, rowmax, gmax, f, hrow, nties):
    mesh = plsc.VectorSubcoreMesh(
        core_axis_name="c", subcore_axis_name="s",
        num_cores=_NCORES, num_subcores=_NSUB,
    )
    kfun = pl.kernel(
        _sc_finish_body,
        out_type=jax.ShapeDtypeStruct((_B, _NS * _C), jnp.float32),
        mesh=mesh,
        scratch_types=[
            pltpu.VMEM((_MAPS_PER_W, _H), jnp.float32),
            pltpu.VMEM((_MAPS_PER_W + 16,), jnp.float32),
            pltpu.VMEM((_MAPS_PER_W + 16,), jnp.float32),
            pltpu.VMEM((_MAPS_PER_W + 16,), jnp.float32),
            pltpu.VMEM((8, _W), jnp.float32),
            pltpu.VMEM((_H + 16,), jnp.float32),
            pltpu.VMEM((4 * _H,), jnp.float32),
            pltpu.VMEM((_C,), jnp.float32),
            pltpu.VMEM((_NS * _C,), jnp.float32),
            pltpu.VMEM_SHARED((_NSUB * _H,), jnp.float32),
        ],
    )
    return kfun(x3, rowmax, gmax, f, hrow, nties)


def kernel(x):
    B, C, H, W = x.shape
    x3 = x.reshape(B * C, H, W)
    CB = 16
    rowmax, gmax3, f3, hrow3, nties3 = pl.pallas_call(
        _stats_body,
        grid=(B * C // CB,),
        in_specs=[pl.BlockSpec((CB, H, W), lambda g: (g, 0, 0))],
        out_specs=[
            pl.BlockSpec((CB, H), lambda g: (g, 0)),
            pl.BlockSpec((1, 1, CB), lambda g: (g, 0, 0)),
            pl.BlockSpec((1, 1, CB), lambda g: (g, 0, 0)),
            pl.BlockSpec((1, 1, CB), lambda g: (g, 0, 0)),
            pl.BlockSpec((1, 1, CB), lambda g: (g, 0, 0)),
        ],
        out_shape=[
            jax.ShapeDtypeStruct((B * C, H), jnp.float32),
            jax.ShapeDtypeStruct((B * C // CB, 1, CB), jnp.float32),
            jax.ShapeDtypeStruct((B * C // CB, 1, CB), jnp.float32),
            jax.ShapeDtypeStruct((B * C // CB, 1, CB), jnp.float32),
            jax.ShapeDtypeStruct((B * C // CB, 1, CB), jnp.float32),
        ],
    )(x3)
    return _sc_finish(
        x.reshape(B * C * H // 8, 8, W),
        rowmax,
        gmax3.reshape(B * C),
        f3.reshape(B * C),
        hrow3.reshape(B * C),
        nties3.reshape(B * C),
    )


# speedup vs baseline: 1.7660x; 1.1188x over previous
import jax
import jax.numpy as jnp
from jax import lax
from jax.experimental import pallas as pl
from jax.experimental.pallas import tpu as pltpu
from jax.experimental.pallas import tpu_sc as plsc

_NS = 8
_B, _C, _H, _W = 8, 96, 224, 224
_NCHUNK = _H // 16
_NCORES = 2
_NSUB = 16
_MAPS_PER_W = (_B * _C) // (_NCORES * _NSUB)


def _stats_body(x_ref, rowmax_ref, gmax_ref, f_ref, hrow_ref, nties_ref):
    cb = x_ref.shape[0]
    facc = jnp.zeros((cb, 8), jnp.float32)
    for t in range(_H // 8):
        xt = x_ref[:, pl.ds(t * 8, 8), :]
        rowmax_ref[:, pl.ds(t * 8, 8)] = jnp.max(xt, axis=2)
        facc = facc + jnp.sum(xt, axis=2)
    f_ref[...] = (jnp.sum(facc, axis=1) / _W).reshape(1, 1, -1)
    rowmax = rowmax_ref[...]
    gmax = jnp.max(rowmax, axis=1)
    gmax_ref[...] = gmax.reshape(1, 1, -1)
    hrow_ref[...] = (
        jnp.argmax(rowmax, axis=1).astype(jnp.float32).reshape(1, 1, -1)
    )
    nties_ref[...] = jnp.sum(
        jnp.where(rowmax == gmax[:, None], 1.0, 0.0), axis=1
    ).reshape(1, 1, -1)


def _splat_f(v):
    return jnp.full((16,), v, jnp.float32)


def _lane_sum(v, iota):
    for s in (1, 2, 4, 8):
        v = v + jnp.take(v, iota ^ s, mode="wrap")
    return v


def _lane_prefix(v, iota, zero):
    for s in (1, 2, 4, 8):
        w = jnp.take(v, jnp.maximum(iota - s, 0), mode="wrap")
        v = v + jnp.where(iota >= s, w, zero)
    return v


def _lane_min(v, iota):
    for s in (1, 2, 4, 8):
        v = jnp.minimum(v, jnp.take(v, iota ^ s, mode="wrap"))
    return v


def _lane_max(v, iota):
    for s in (1, 2, 4, 8):
        v = jnp.maximum(v, jnp.take(v, iota ^ s, mode="wrap"))
    return v


def _sc_finish_body(
    x3_hbm, rowmax_hbm, gmax_hbm, f_hbm, hrow_hbm, nties_hbm, out_hbm,
    rm_buf, gm_buf, hrow_buf, nties_buf, rows_v,
    hist, hist4, fv, outb, shared,
):
    cid = lax.axis_index("c")
    sid = lax.axis_index("s")
    g0 = cid * (_NSUB * _MAPS_PER_W) + sid * _MAPS_PER_W

    pltpu.sync_copy(
        gmax_hbm.at[pl.ds(g0, _MAPS_PER_W)], gm_buf.at[pl.ds(0, _MAPS_PER_W)]
    )
    pltpu.sync_copy(
        hrow_hbm.at[pl.ds(g0, _MAPS_PER_W)], hrow_buf.at[pl.ds(0, _MAPS_PER_W)]
    )
    pltpu.sync_copy(
        nties_hbm.at[pl.ds(g0, _MAPS_PER_W)],
        nties_buf.at[pl.ds(0, _MAPS_PER_W)],
    )

    zero16f = jnp.zeros((16,), jnp.float32)
    one16f = jnp.ones((16,), jnp.float32)
    for i in range(_NCHUNK + 1):
        hist[pl.ds(i * 16, 16)] = zero16f

    iota16 = lax.iota(jnp.int32, 16)
    iota_f = iota16.astype(jnp.float32)
    big16f = jnp.full((16,), 1.0e9, jnp.float32)

    nta = jnp.maximum(nties_buf[pl.ds(0, 16)], nties_buf[pl.ds(8, 16)])
    any_slow = _lane_max(nta, iota16)[0] > 1.0

    @pl.when(any_slow)
    def _load_rowmax():
        pltpu.sync_copy(rowmax_hbm.at[pl.ds(g0, _MAPS_PER_W)], rm_buf)

    def _count_row(h_i, gm_v, tile_i):
        pltpu.sync_copy(x3_hbm.at[tile_i], rows_v)
        hsub = lax.rem(h_i, 8)
        cnt = zero16f
        for i in range(_NCHUNK):
            xc = rows_v[hsub, pl.ds(i * 16, 16)]
            cnt = cnt + jnp.where(xc == gm_v, one16f, zero16f)
        tot = _lane_sum(cnt, iota16)
        hv = hist[pl.ds(h_i, 16)]
        hist[pl.ds(h_i, 16)] = hv + jnp.where(iota16 == 0, tot, 0.0)

    def map_body(m, _):
        gm_v = _splat_f(gm_buf[pl.ds(m, 16)][0])
        nt = nties_buf[pl.ds(m, 16)][0]
        tbase = (g0 + m) * (_H // 8)

        @pl.when(nt == 1.0)
        def _fast():
            h_i = hrow_buf[pl.ds(m, 16)][0].astype(jnp.int32)
            _count_row(h_i, gm_v, tbase + lax.div(h_i, 8))

        @pl.when(nt > 1.0)
        def _slow():
            for i in range(_NCHUNK):
                rm_c = rm_buf[m, pl.ds(i * 16, 16)]
                mi = jnp.where(rm_c == gm_v, one16f, zero16f)
                nh = _lane_sum(mi, iota16)[0].astype(jnp.int32)

                def hit_body(_hh, mi2):
                    lane = jnp.where(mi2 > 0.0, iota_f, big16f)
                    h0 = _lane_min(lane, iota16)[0]
                    h_i = h0.astype(jnp.int32) + i * 16
                    _count_row(h_i, gm_v, tbase + lax.div(h_i, 8))
                    return jnp.where(iota_f == _splat_f(h0), zero16f, mi2)

                lax.fori_loop(0, nh, hit_body, mi)

        return _

    lax.fori_loop(0, _MAPS_PER_W, map_body, 0)

    pltpu.sync_copy(hist.at[pl.ds(0, _H)], shared.at[pl.ds(sid * _H, _H)])
    plsc.subcore_barrier()

    @pl.when(sid < _B // _NCORES)
    def _finalize():
        b = cid * (_B // _NCORES) + sid
        pltpu.sync_copy(shared.at[pl.ds(sid * 4 * _H, 4 * _H)], hist4)
        pltpu.sync_copy(f_hbm.at[pl.ds(b * _C, _C)], fv)

        jcnt = [zero16f] * _NS
        thr = [_splat_f(float((k * _C) // _NS)) for k in range(_NS)]

        def chunk_body(i, carry):
            carry_s, *jc = carry
            v = (
                hist4[pl.ds(i * 16, 16)]
                + hist4[pl.ds(_H + i * 16, 16)]
                + hist4[pl.ds(2 * _H + i * 16, 16)]
                + hist4[pl.ds(3 * _H + i * 16, 16)]
            )
            cs = _lane_prefix(v, iota16, zero16f)
            hexc = cs - v + _splat_f(carry_s)
            jc = list(jc)
            for k in range(1, _NS):
                jc[k] = jc[k] + jnp.where(hexc <= thr[k], one16f, zero16f)
            carry_s = carry_s + cs[15]
            return (carry_s, *jc)

        res = lax.fori_loop(
            0, _NCHUNK, chunk_body, (jnp.float32(0.0), *jcnt)
        )
        jcnt = [
            _lane_sum(v, iota16) if k else v for k, v in enumerate(res[1:])
        ]

        one_f = jnp.ones((16,), jnp.float32)
        prev = jnp.zeros((16,), jnp.float32)
        ok = jnp.ones((16,), jnp.bool_)
        hk = [jnp.zeros((16,), jnp.float32)]
        for k in range(1, _NS):
            jk = jcnt[k] - one_f
            good = (jk >= one_f) & (jk <= float(_H - 2)) & (jk > prev)
            ok = ok & good
            hk.append(jnp.where(ok, jk, 0.0))
            prev = jk
        hk.append(_splat_f(float(_H)))

        for s in range(_NS):
            d_v = hk[s + 1] - hk[s]
            for ci in range(_C // 16):
                outb[pl.ds(s * _C + ci * 16, 16)] = fv[pl.ds(ci * 16, 16)] / d_v

        pltpu.sync_copy(outb, out_hbm.at[b])


def _sc_finish(x3, rowmax, gmax, f, hrow, nties):
    mesh = plsc.VectorSubcoreMesh(
        core_axis_name="c", subcore_axis_name="s",
        num_cores=_NCORES, num_subcores=_NSUB,
    )
    kfun = pl.kernel(
        _sc_finish_body,
        out_type=jax.ShapeDtypeStruct((_B, _NS * _C), jnp.float32),
        mesh=mesh,
        scratch_types=[
            pltpu.VMEM((_MAPS_PER_W, _H), jnp.float32),
            pltpu.VMEM((_MAPS_PER_W + 16,), jnp.float32),
            pltpu.VMEM((_MAPS_PER_W + 16,), jnp.float32),
            pltpu.VMEM((_MAPS_PER_W + 16,), jnp.float32),
            pltpu.VMEM((8, _W), jnp.float32),
            pltpu.VMEM((_H + 16,), jnp.float32),
            pltpu.VMEM((4 * _H,), jnp.float32),
            pltpu.VMEM((_C,), jnp.float32),
            pltpu.VMEM((_NS * _C,), jnp.float32),
            pltpu.VMEM_SHARED((_NSUB * _H,), jnp.float32),
        ],
    )
    return kfun(x3, rowmax, gmax, f, hrow, nties)


def kernel(x):
    B, C, H, W = x.shape
    x3 = x.reshape(B * C, H, W)
    CB = 32
    rowmax, gmax3, f3, hrow3, nties3 = pl.pallas_call(
        _stats_body,
        grid=(B * C // CB,),
        in_specs=[pl.BlockSpec((CB, H, W), lambda g: (g, 0, 0))],
        out_specs=[
            pl.BlockSpec((CB, H), lambda g: (g, 0)),
            pl.BlockSpec((1, 1, CB), lambda g: (g, 0, 0)),
            pl.BlockSpec((1, 1, CB), lambda g: (g, 0, 0)),
            pl.BlockSpec((1, 1, CB), lambda g: (g, 0, 0)),
            pl.BlockSpec((1, 1, CB), lambda g: (g, 0, 0)),
        ],
        out_shape=[
            jax.ShapeDtypeStruct((B * C, H), jnp.float32),
            jax.ShapeDtypeStruct((B * C // CB, 1, CB), jnp.float32),
            jax.ShapeDtypeStruct((B * C // CB, 1, CB), jnp.float32),
            jax.ShapeDtypeStruct((B * C // CB, 1, CB), jnp.float32),
            jax.ShapeDtypeStruct((B * C // CB, 1, CB), jnp.float32),
        ],
    )(x3)
    return _sc_finish(
        x.reshape(B * C * H // 8, 8, W),
        rowmax,
        gmax3.reshape(B * C),
        f3.reshape(B * C),
        hrow3.reshape(B * C),
        nties3.reshape(B * C),
    )


# speedup vs baseline: 1.8899x; 1.0702x over previous
import jax
import jax.numpy as jnp
from jax import lax
from jax.experimental import pallas as pl
from jax.experimental.pallas import tpu as pltpu
from jax.experimental.pallas import tpu_sc as plsc

_NS = 8
_B, _C, _H, _W = 8, 96, 224, 224
_NCHUNK = _H // 16
_NCORES = 2
_NSUB = 16
_MAPS_PER_W = (_B * _C) // (_NCORES * _NSUB)


def _stats_body(x_ref, rowmax_ref, gmax_ref, f_ref, hrow_ref, nties_ref):
    cb = x_ref.shape[0]
    facc = jnp.zeros((cb, 8), jnp.float32)
    for t in range(_H // 8):
        xt = x_ref[:, pl.ds(t * 8, 8), :]
        rowmax_ref[:, pl.ds(t * 8, 8)] = jnp.max(xt, axis=2)
        facc = facc + jnp.sum(xt, axis=2)
    f_ref[...] = (jnp.sum(facc, axis=1) / _W).reshape(1, 1, -1)
    rowmax = rowmax_ref[...]
    gmax = jnp.max(rowmax, axis=1)
    gmax_ref[...] = gmax.reshape(1, 1, -1)
    hrow_ref[...] = (
        jnp.argmax(rowmax, axis=1).astype(jnp.float32).reshape(1, 1, -1)
    )
    nties_ref[...] = jnp.sum(
        jnp.where(rowmax == gmax[:, None], 1.0, 0.0), axis=1
    ).reshape(1, 1, -1)


def _splat_f(v):
    return jnp.full((16,), v, jnp.float32)


def _lane_sum(v, iota):
    for s in (1, 2, 4, 8):
        v = v + jnp.take(v, iota ^ s, mode="wrap")
    return v


def _lane_prefix(v, iota, zero):
    for s in (1, 2, 4, 8):
        w = jnp.take(v, jnp.maximum(iota - s, 0), mode="wrap")
        v = v + jnp.where(iota >= s, w, zero)
    return v


def _lane_min(v, iota):
    for s in (1, 2, 4, 8):
        v = jnp.minimum(v, jnp.take(v, iota ^ s, mode="wrap"))
    return v


def _lane_max(v, iota):
    for s in (1, 2, 4, 8):
        v = jnp.maximum(v, jnp.take(v, iota ^ s, mode="wrap"))
    return v


def _sc_finish_body(
    x3_hbm, rowmax_hbm, gmax_hbm, f_hbm, hrow_hbm, nties_hbm, out_hbm,
    rm_buf, gm_buf, hrow_buf, nties_buf, rows_v,
    hist, hist4, fv, outb, shared,
):
    cid = lax.axis_index("c")
    sid = lax.axis_index("s")
    g0 = cid * (_NSUB * _MAPS_PER_W) + sid * _MAPS_PER_W

    pltpu.sync_copy(
        gmax_hbm.at[pl.ds(g0, _MAPS_PER_W)], gm_buf.at[pl.ds(0, _MAPS_PER_W)]
    )
    pltpu.sync_copy(
        hrow_hbm.at[pl.ds(g0, _MAPS_PER_W)], hrow_buf.at[pl.ds(0, _MAPS_PER_W)]
    )
    pltpu.sync_copy(
        nties_hbm.at[pl.ds(g0, _MAPS_PER_W)],
        nties_buf.at[pl.ds(0, _MAPS_PER_W)],
    )

    zero16f = jnp.zeros((16,), jnp.float32)
    one16f = jnp.ones((16,), jnp.float32)
    for i in range(_NCHUNK + 1):
        hist[pl.ds(i * 16, 16)] = zero16f

    iota16 = lax.iota(jnp.int32, 16)
    iota_f = iota16.astype(jnp.float32)
    big16f = jnp.full((16,), 1.0e9, jnp.float32)

    nta = jnp.maximum(nties_buf[pl.ds(0, 16)], nties_buf[pl.ds(8, 16)])
    any_slow = _lane_max(nta, iota16)[0] > 1.0

    @pl.when(any_slow)
    def _load_rowmax():
        pltpu.sync_copy(rowmax_hbm.at[pl.ds(g0, _MAPS_PER_W)], rm_buf)

    def _count_row(h_i, gm_v, tile_i):
        pltpu.sync_copy(x3_hbm.at[tile_i], rows_v)
        hsub = lax.rem(h_i, 8)
        cnt = zero16f
        for i in range(_NCHUNK):
            xc = rows_v[hsub, pl.ds(i * 16, 16)]
            cnt = cnt + jnp.where(xc == gm_v, one16f, zero16f)
        tot = _lane_sum(cnt, iota16)
        hv = hist[pl.ds(h_i, 16)]
        hist[pl.ds(h_i, 16)] = hv + jnp.where(iota16 == 0, tot, 0.0)

    def map_body(m, _):
        gm_v = _splat_f(gm_buf[pl.ds(m, 16)][0])
        nt = nties_buf[pl.ds(m, 16)][0]
        tbase = (g0 + m) * (_H // 8)

        @pl.when(nt == 1.0)
        def _fast():
            h_i = hrow_buf[pl.ds(m, 16)][0].astype(jnp.int32)
            _count_row(h_i, gm_v, tbase + lax.div(h_i, 8))

        @pl.when(nt > 1.0)
        def _slow():
            for i in range(_NCHUNK):
                rm_c = rm_buf[m, pl.ds(i * 16, 16)]
                mi = jnp.where(rm_c == gm_v, one16f, zero16f)
                nh = _lane_sum(mi, iota16)[0].astype(jnp.int32)

                def hit_body(_hh, mi2):
                    lane = jnp.where(mi2 > 0.0, iota_f, big16f)
                    h0 = _lane_min(lane, iota16)[0]
                    h_i = h0.astype(jnp.int32) + i * 16
                    _count_row(h_i, gm_v, tbase + lax.div(h_i, 8))
                    return jnp.where(iota_f == _splat_f(h0), zero16f, mi2)

                lax.fori_loop(0, nh, hit_body, mi)

        return _

    lax.fori_loop(0, _MAPS_PER_W, map_body, 0)

    pltpu.sync_copy(hist.at[pl.ds(0, _H)], shared.at[pl.ds(sid * _H, _H)])
    plsc.subcore_barrier()

    @pl.when(sid < _B // _NCORES)
    def _finalize():
        b = cid * (_B // _NCORES) + sid
        pltpu.sync_copy(shared.at[pl.ds(sid * 4 * _H, 4 * _H)], hist4)
        pltpu.sync_copy(f_hbm.at[pl.ds(b * _C, _C)], fv)

        jcnt = [zero16f] * _NS
        thr = [_splat_f(float((k * _C) // _NS)) for k in range(_NS)]

        def chunk_body(i, carry):
            carry_s, *jc = carry
            v = (
                hist4[pl.ds(i * 16, 16)]
                + hist4[pl.ds(_H + i * 16, 16)]
                + hist4[pl.ds(2 * _H + i * 16, 16)]
                + hist4[pl.ds(3 * _H + i * 16, 16)]
            )
            cs = _lane_prefix(v, iota16, zero16f)
            hexc = cs - v + _splat_f(carry_s)
            jc = list(jc)
            for k in range(1, _NS):
                jc[k] = jc[k] + jnp.where(hexc <= thr[k], one16f, zero16f)
            carry_s = carry_s + cs[15]
            return (carry_s, *jc)

        res = lax.fori_loop(
            0, _NCHUNK, chunk_body, (jnp.float32(0.0), *jcnt)
        )
        jcnt = [
            _lane_sum(v, iota16) if k else v for k, v in enumerate(res[1:])
        ]

        one_f = jnp.ones((16,), jnp.float32)
        prev = jnp.zeros((16,), jnp.float32)
        ok = jnp.ones((16,), jnp.bool_)
        hk = [jnp.zeros((16,), jnp.float32)]
        for k in range(1, _NS):
            jk = jcnt[k] - one_f
            good = (jk >= one_f) & (jk <= float(_H - 2)) & (jk > prev)
            ok = ok & good
            hk.append(jnp.where(ok, jk, 0.0))
            prev = jk
        hk.append(_splat_f(float(_H)))

        for s in range(_NS):
            d_v = hk[s + 1] - hk[s]
            for ci in range(_C // 16):
                outb[pl.ds(s * _C + ci * 16, 16)] = fv[pl.ds(ci * 16, 16)] / d_v

        pltpu.sync_copy(outb, out_hbm.at[b])


def _sc_finish(x3, rowmax, gmax, f, hrow, nties):
    mesh = plsc.VectorSubcoreMesh(
        core_axis_name="c", subcore_axis_name="s",
        num_cores=_NCORES, num_subcores=_NSUB,
    )
    kfun = pl.kernel(
        _sc_finish_body,
        out_type=jax.ShapeDtypeStruct((_B, _NS * _C), jnp.float32),
        mesh=mesh,
        scratch_types=[
            pltpu.VMEM((_MAPS_PER_W, _H), jnp.float32),
            pltpu.VMEM((_MAPS_PER_W + 16,), jnp.float32),
            pltpu.VMEM((_MAPS_PER_W + 16,), jnp.float32),
            pltpu.VMEM((_MAPS_PER_W + 16,), jnp.float32),
            pltpu.VMEM((8, _W), jnp.float32),
            pltpu.VMEM((_H + 16,), jnp.float32),
            pltpu.VMEM((4 * _H,), jnp.float32),
            pltpu.VMEM((_C,), jnp.float32),
            pltpu.VMEM((_NS * _C,), jnp.float32),
            pltpu.VMEM_SHARED((_NSUB * _H,), jnp.float32),
        ],
    )
    return kfun(x3, rowmax, gmax, f, hrow, nties)


def kernel(x):
    B, C, H, W = x.shape
    x3 = x.reshape(B * C, H, W)
    CB = 64
    rowmax, gmax3, f3, hrow3, nties3 = pl.pallas_call(
        _stats_body,
        grid=(B * C // CB,),
        in_specs=[pl.BlockSpec((CB, H, W), lambda g: (g, 0, 0))],
        out_specs=[
            pl.BlockSpec((CB, H), lambda g: (g, 0)),
            pl.BlockSpec((1, 1, CB), lambda g: (g, 0, 0)),
            pl.BlockSpec((1, 1, CB), lambda g: (g, 0, 0)),
            pl.BlockSpec((1, 1, CB), lambda g: (g, 0, 0)),
            pl.BlockSpec((1, 1, CB), lambda g: (g, 0, 0)),
        ],
        out_shape=[
            jax.ShapeDtypeStruct((B * C, H), jnp.float32),
            jax.ShapeDtypeStruct((B * C // CB, 1, CB), jnp.float32),
            jax.ShapeDtypeStruct((B * C // CB, 1, CB), jnp.float32),
            jax.ShapeDtypeStruct((B * C // CB, 1, CB), jnp.float32),
            jax.ShapeDtypeStruct((B * C // CB, 1, CB), jnp.float32),
        ],
    )(x3)
    return _sc_finish(
        x.reshape(B * C * H // 8, 8, W),
        rowmax,
        gmax3.reshape(B * C),
        f3.reshape(B * C),
        hrow3.reshape(B * C),
        nties3.reshape(B * C),
    )
